# baseline (speedup 1.0000x reference)
import functools

import numpy as np
import jax
import jax.numpy as jnp
from jax import lax
from jax.experimental import pallas as pl
from jax.experimental.pallas import tpu as pltpu
from jax.experimental.pallas import tpu_sc as plsc

_RANDI = np.array([0, 3, 5, 7, 8, 11, 12, 15, 16, 19, 20, 23, 24, 27, 29, 30,
                   33, 35, 37, 39, 41, 43, 44, 47, 49, 51, 53, 54, 56, 59, 60,
                   63], dtype=np.int32)
_RANDJ = np.array([1, 2, 4, 6, 9, 10, 13, 14, 16, 19, 20, 22, 24, 27, 29, 30,
                   33, 34, 37, 39, 41, 42, 44, 46, 48, 50, 52, 54, 56, 59, 60,
                   62], dtype=np.int32)

_B, _C, _H, _W = 1024, 16, 64, 64
_NIN = _C * _H * _W
_NOUT = _C * 32 * 32
_NW = 32
_RPW = _NOUT // _NW
_K = 32
_NCH = _RPW // _K
_NBUF = 3
_LOOK = 2

_SRC_IDX = ((np.arange(_C)[:, None, None] * _H + _RANDJ[None, :, None]) * _W
            + _RANDI[None, None, :]).reshape(-1).astype(np.int32)

_mesh = plsc.VectorSubcoreMesh(core_axis_name="c", subcore_axis_name="s")


@functools.partial(
    pl.kernel,
    out_type=jax.ShapeDtypeStruct((_NOUT, _B), jnp.float32),
    mesh=_mesh,
    compiler_params=pltpu.CompilerParams(needs_layout_passes=False,
                                         use_tc_tiling_on_sc=True),
    scratch_types=[
        pltpu.VMEM((_RPW,), jnp.int32),
        pltpu.VMEM((_NBUF, _K, _B), jnp.float32),
        [pltpu.SemaphoreType.DMA] * _NBUF,
        [pltpu.SemaphoreType.DMA] * _NBUF,
    ],
)
def _sc_gather(table, idxs, out, idx_all, rows, gsems, osems):
    wid = lax.axis_index("s") * 2 + lax.axis_index("c")
    base = pl.multiple_of(wid * _RPW, _RPW)

    pltpu.sync_copy(idxs.at[pl.ds(base, _RPW)], idx_all)

    def gfire(k):
        pltpu.async_copy(table.at[idx_all.at[pl.ds(k * _K, _K)]],
                         rows.at[k % _NBUF], gsems[k % _NBUF])

    def gwait(k):
        pltpu.make_async_copy(table.at[idx_all.at[pl.ds(k * _K, _K)]],
                              rows.at[k % _NBUF], gsems[k % _NBUF]).wait()

    def odesc(k):
        dst = out.at[pl.ds(pl.multiple_of(base + k * _K, _K), _K)]
        return pltpu.make_async_copy(rows.at[k % _NBUF], dst, osems[k % _NBUF])

    for k in range(_LOOK):
        gfire(k)
    for k in range(_NCH):
        gwait(k)
        odesc(k).start()
        if k + _LOOK < _NCH:
            if k - (_NBUF - _LOOK) >= 0:
                odesc(k - (_NBUF - _LOOK)).wait()
            gfire(k + _LOOK)
    for k in range(_NCH - _NBUF, _NCH):
        odesc(k).wait()


def kernel(x):
    table = x.transpose(1, 2, 3, 0).reshape(_NIN, _B)
    idxs = jnp.asarray(_SRC_IDX)
    out = _sc_gather(table, idxs)
    return out.reshape(_C, 32, 32, _B).transpose(3, 0, 1, 2)


# speedup vs baseline: 1.0138x; 1.0138x over previous
import functools

import numpy as np
import jax
import jax.numpy as jnp
from jax import lax
from jax.experimental import pallas as pl
from jax.experimental.pallas import tpu as pltpu
from jax.experimental.pallas import tpu_sc as plsc

_RANDI = np.array([0, 3, 5, 7, 8, 11, 12, 15, 16, 19, 20, 23, 24, 27, 29, 30,
                   33, 35, 37, 39, 41, 43, 44, 47, 49, 51, 53, 54, 56, 59, 60,
                   63], dtype=np.int32)
_RANDJ = np.array([1, 2, 4, 6, 9, 10, 13, 14, 16, 19, 20, 22, 24, 27, 29, 30,
                   33, 34, 37, 39, 41, 42, 44, 46, 48, 50, 52, 54, 56, 59, 60,
                   62], dtype=np.int32)

_B, _C, _H, _W = 1024, 16, 64, 64
_NIN = _C * _H * _W
_NOUT = _C * 32 * 32
_NW = 32
_RPW = _NOUT // _NW
_K = 8
_NCH = _RPW // _K
_NBUF = 12
_LOOK = 8

_SRC_IDX = ((np.arange(_C)[:, None, None] * _H + _RANDJ[None, :, None]) * _W
            + _RANDI[None, None, :]).reshape(-1).astype(np.int32)

_mesh = plsc.VectorSubcoreMesh(core_axis_name="c", subcore_axis_name="s")


@functools.partial(
    pl.kernel,
    out_type=jax.ShapeDtypeStruct((_NOUT, _B), jnp.float32),
    mesh=_mesh,
    compiler_params=pltpu.CompilerParams(needs_layout_passes=False,
                                         use_tc_tiling_on_sc=True),
    scratch_types=[
        pltpu.VMEM((_RPW,), jnp.int32),
        pltpu.VMEM((_NBUF, _K, _B), jnp.float32),
        [pltpu.SemaphoreType.DMA] * _NBUF,
        [pltpu.SemaphoreType.DMA] * _NBUF,
    ],
)
def _sc_gather(table, idxs, out, idx_all, rows, gsems, osems):
    wid = lax.axis_index("s") * 2 + lax.axis_index("c")
    base = pl.multiple_of(wid * _RPW, _RPW)

    pltpu.sync_copy(idxs.at[pl.ds(base, _RPW)], idx_all)

    def gfire(k):
        pltpu.async_copy(table.at[idx_all.at[pl.ds(k * _K, _K)]],
                         rows.at[k % _NBUF], gsems[k % _NBUF])

    def gwait(k):
        pltpu.make_async_copy(table.at[idx_all.at[pl.ds(k * _K, _K)]],
                              rows.at[k % _NBUF], gsems[k % _NBUF]).wait()

    def odesc(k):
        dst = out.at[pl.ds(pl.multiple_of(base + k * _K, _K), _K)]
        return pltpu.make_async_copy(rows.at[k % _NBUF], dst, osems[k % _NBUF])

    for k in range(_LOOK):
        gfire(k)
    for k in range(_NCH):
        gwait(k)
        odesc(k).start()
        if k + _LOOK < _NCH:
            if k - (_NBUF - _LOOK) >= 0:
                odesc(k - (_NBUF - _LOOK)).wait()
            gfire(k + _LOOK)
    for k in range(_NCH - _NBUF, _NCH):
        odesc(k).wait()


def kernel(x):
    table = x.transpose(1, 2, 3, 0).reshape(_NIN, _B)
    idxs = jnp.asarray(_SRC_IDX)
    out = _sc_gather(table, idxs)
    return out.reshape(_C, 32, 32, _B).transpose(3, 0, 1, 2)


# speedup vs baseline: 1.0289x; 1.0149x over previous
import functools

import numpy as np
import jax
import jax.numpy as jnp
from jax import lax
from jax.experimental import pallas as pl
from jax.experimental.pallas import tpu as pltpu
from jax.experimental.pallas import tpu_sc as plsc

_RANDI = np.array([0, 3, 5, 7, 8, 11, 12, 15, 16, 19, 20, 23, 24, 27, 29, 30,
                   33, 35, 37, 39, 41, 43, 44, 47, 49, 51, 53, 54, 56, 59, 60,
                   63], dtype=np.int32)
_RANDJ = np.array([1, 2, 4, 6, 9, 10, 13, 14, 16, 19, 20, 22, 24, 27, 29, 30,
                   33, 34, 37, 39, 41, 42, 44, 46, 48, 50, 52, 54, 56, 59, 60,
                   62], dtype=np.int32)

_B, _C, _H, _W = 1024, 16, 64, 64
_NIN = _C * _H * _W
_NOUT = _C * 32 * 32
_NW = 32
_RPW = _NOUT // _NW
_K = 16
_NCH = _RPW // _K
_NBUF = 7
_LOOK = 5

_SRC_IDX = ((np.arange(_C)[:, None, None] * _H + _RANDJ[None, :, None]) * _W
            + _RANDI[None, None, :]).reshape(-1).astype(np.int32)

_mesh = plsc.VectorSubcoreMesh(core_axis_name="c", subcore_axis_name="s")


@functools.partial(
    pl.kernel,
    out_type=jax.ShapeDtypeStruct((_NOUT, _B), jnp.float32),
    mesh=_mesh,
    compiler_params=pltpu.CompilerParams(needs_layout_passes=False,
                                         use_tc_tiling_on_sc=True),
    scratch_types=[
        pltpu.VMEM((_RPW,), jnp.int32),
        pltpu.VMEM((_NBUF, _K, _B), jnp.float32),
        [pltpu.SemaphoreType.DMA] * _NBUF,
        [pltpu.SemaphoreType.DMA] * _NBUF,
    ],
)
def _sc_gather(table, idxs, out, idx_all, rows, gsems, osems):
    wid = lax.axis_index("s") * 2 + lax.axis_index("c")
    base = pl.multiple_of(wid * _RPW, _RPW)

    pltpu.sync_copy(idxs.at[pl.ds(base, _RPW)], idx_all)

    def gfire(k):
        pltpu.async_copy(table.at[idx_all.at[pl.ds(k * _K, _K)]],
                         rows.at[k % _NBUF], gsems[k % _NBUF])

    def gwait(k):
        pltpu.make_async_copy(table.at[idx_all.at[pl.ds(k * _K, _K)]],
                              rows.at[k % _NBUF], gsems[k % _NBUF]).wait()

    def odesc(k):
        dst = out.at[pl.ds(pl.multiple_of(base + k * _K, _K), _K)]
        return pltpu.make_async_copy(rows.at[k % _NBUF], dst, osems[k % _NBUF])

    for k in range(_LOOK):
        gfire(k)
    for k in range(_NCH):
        gwait(k)
        odesc(k).start()
        if k + _LOOK < _NCH:
            if k - (_NBUF - _LOOK) >= 0:
                odesc(k - (_NBUF - _LOOK)).wait()
            gfire(k + _LOOK)
    for k in range(_NCH - _NBUF, _NCH):
        odesc(k).wait()


def kernel(x):
    table = x.transpose(1, 2, 3, 0).reshape(_NIN, _B)
    idxs = jnp.asarray(_SRC_IDX)
    out = _sc_gather(table, idxs)
    return out.reshape(_C, 32, 32, _B).transpose(3, 0, 1, 2)
